# baseline (speedup 1.0000x reference)
import functools

import jax
import jax.numpy as jnp
from jax import lax
from jax.experimental import pallas as pl
from jax.experimental.pallas import tpu as pltpu
from jax.experimental.pallas import tpu_sc as plsc

EPS = 1e-5
D = 128
_INV_D = 1.0 / D


_ROWS_BLK = 20000


def _ln_body(tab_ref, gamma_ref, beta_ref, out_ref):
    t = tab_ref[...]
    ones_over_d = jnp.full((D, D), _INV_D, dtype=jnp.float32)
    mean = jax.lax.dot(t, ones_over_d)
    m2 = jax.lax.dot(t * t, ones_over_d)
    var = m2 - mean * mean
    out_ref[...] = (t - mean) * lax.rsqrt(var + EPS) * gamma_ref[...] + beta_ref[...]


def _ln_table(table, gamma, beta):
    n_rows = table.shape[0]
    grid = n_rows // _ROWS_BLK
    return pl.pallas_call(
        _ln_body,
        grid=(grid,),
        in_specs=[
            pl.BlockSpec((_ROWS_BLK, D), lambda i: (i, 0)),
            pl.BlockSpec((1, D), lambda i: (0, 0)),
            pl.BlockSpec((1, D), lambda i: (0, 0)),
        ],
        out_specs=pl.BlockSpec((_ROWS_BLK, D), lambda i: (i, 0)),
        out_shape=jax.ShapeDtypeStruct((n_rows, D), jnp.float32),
    )(table, gamma.reshape(1, D), beta.reshape(1, D))


_NC, _NS = 2, 16
_NW = _NC * _NS
_K = 400


def _make_gather(B):
    b_per_w = B // _NW
    nchunk = b_per_w // _K
    mesh = plsc.VectorSubcoreMesh(core_axis_name="c", subcore_axis_name="s")

    @functools.partial(
        pl.kernel,
        mesh=mesh,
        out_type=jax.ShapeDtypeStruct((B, D), jnp.float32),
        scratch_types=[
            pltpu.VMEM((b_per_w,), jnp.int32),
            pltpu.VMEM((_K, D), jnp.float32),
            pltpu.VMEM((_K, D), jnp.float32),
            pltpu.SemaphoreType.DMA,
            pltpu.SemaphoreType.DMA,
            pltpu.SemaphoreType.DMA,
            pltpu.SemaphoreType.DMA,
        ],
    )
    def gather(tab_hbm, idx_hbm, out_hbm, idx_v, rows0, rows1, gs0, gs1, os0, os1):
        wid = lax.axis_index("s") * _NC + lax.axis_index("c")
        base = wid * b_per_w
        pltpu.sync_copy(idx_hbm.at[pl.ds(base, b_per_w)], idx_v)
        bufs = (rows0, rows1)
        gsems = (gs0, gs1)
        osems = (os0, os1)
        h_in = [None, None]
        h_out = [None, None]
        h_in[0] = pltpu.async_copy(
            tab_hbm.at[idx_v.at[pl.ds(0, _K)]], bufs[0], gsems[0]
        )
        for g in range(nchunk):
            b = g % 2
            if g + 1 < nchunk:
                b2 = (g + 1) % 2
                if h_out[b2] is not None:
                    h_out[b2].wait()
                h_in[b2] = pltpu.async_copy(
                    tab_hbm.at[idx_v.at[pl.ds((g + 1) * _K, _K)]],
                    bufs[b2],
                    gsems[b2],
                )
            h_in[b].wait()
            h_out[b] = pltpu.async_copy(
                bufs[b], out_hbm.at[pl.ds(base + g * _K, _K)], osems[b]
            )
        for h in h_out:
            if h is not None:
                h.wait()

    return gather


def kernel(x, table, gamma, beta):
    batch, hist = x.shape
    B = batch * hist
    table_n = _ln_table(table, gamma, beta)
    idx_lmajor = x.T.reshape(B)
    out_flat = _make_gather(B)(table_n, idx_lmajor)
    return out_flat.reshape(hist, batch, D).transpose(1, 0, 2)


# speedup vs baseline: 1.0018x; 1.0018x over previous
import functools

import jax
import jax.numpy as jnp
from jax import lax
from jax.experimental import pallas as pl
from jax.experimental.pallas import tpu as pltpu
from jax.experimental.pallas import tpu_sc as plsc

EPS = 1e-5
D = 128
_INV_D = 1.0 / D


_ROWS_BLK = 10000


def _ln_body(tab_ref, gamma_ref, beta_ref, out_ref):
    t = tab_ref[...]
    ones_over_d = jnp.full((D, D), _INV_D, dtype=jnp.float32)
    mean = jax.lax.dot(t, ones_over_d)
    m2 = jax.lax.dot(t * t, ones_over_d)
    var = m2 - mean * mean
    out_ref[...] = (t - mean) * lax.rsqrt(var + EPS) * gamma_ref[...] + beta_ref[...]


def _ln_table(table, gamma, beta):
    n_rows = table.shape[0]
    grid = n_rows // _ROWS_BLK
    return pl.pallas_call(
        _ln_body,
        grid=(grid,),
        in_specs=[
            pl.BlockSpec((_ROWS_BLK, D), lambda i: (i, 0)),
            pl.BlockSpec((1, D), lambda i: (0, 0)),
            pl.BlockSpec((1, D), lambda i: (0, 0)),
        ],
        out_specs=pl.BlockSpec((_ROWS_BLK, D), lambda i: (i, 0)),
        out_shape=jax.ShapeDtypeStruct((n_rows, D), jnp.float32),
    )(table, gamma.reshape(1, D), beta.reshape(1, D))


_NC, _NS = 2, 16
_NW = _NC * _NS
_K = 400


def _make_gather(B):
    b_per_w = B // _NW
    nchunk = b_per_w // _K
    mesh = plsc.VectorSubcoreMesh(core_axis_name="c", subcore_axis_name="s")

    @functools.partial(
        pl.kernel,
        mesh=mesh,
        out_type=jax.ShapeDtypeStruct((B, D), jnp.float32),
        scratch_types=[
            pltpu.VMEM((b_per_w,), jnp.int32),
            pltpu.VMEM((_K, D), jnp.float32),
            pltpu.VMEM((_K, D), jnp.float32),
            pltpu.SemaphoreType.DMA,
            pltpu.SemaphoreType.DMA,
            pltpu.SemaphoreType.DMA,
            pltpu.SemaphoreType.DMA,
        ],
    )
    def gather(tab_hbm, idx_hbm, out_hbm, idx_v, rows0, rows1, gs0, gs1, os0, os1):
        wid = lax.axis_index("s") * _NC + lax.axis_index("c")
        base = wid * b_per_w
        pltpu.sync_copy(idx_hbm.at[pl.ds(base, b_per_w)], idx_v)
        bufs = (rows0, rows1)
        gsems = (gs0, gs1)
        osems = (os0, os1)
        h_in = [None, None]
        h_out = [None, None]
        h_in[0] = pltpu.async_copy(
            tab_hbm.at[idx_v.at[pl.ds(0, _K)]], bufs[0], gsems[0]
        )
        for g in range(nchunk):
            b = g % 2
            if g + 1 < nchunk:
                b2 = (g + 1) % 2
                if h_out[b2] is not None:
                    h_out[b2].wait()
                h_in[b2] = pltpu.async_copy(
                    tab_hbm.at[idx_v.at[pl.ds((g + 1) * _K, _K)]],
                    bufs[b2],
                    gsems[b2],
                )
            h_in[b].wait()
            h_out[b] = pltpu.async_copy(
                bufs[b], out_hbm.at[pl.ds(base + g * _K, _K)], osems[b]
            )
        for h in h_out:
            if h is not None:
                h.wait()

    return gather


def kernel(x, table, gamma, beta):
    batch, hist = x.shape
    B = batch * hist
    table_n = _ln_table(table, gamma, beta)
    idx_lmajor = x.T.reshape(B)
    out_flat = _make_gather(B)(table_n, idx_lmajor)
    return out_flat.reshape(hist, batch, D).transpose(1, 0, 2)


# speedup vs baseline: 1.0049x; 1.0031x over previous
import functools

import jax
import jax.numpy as jnp
from jax import lax
from jax.experimental import pallas as pl
from jax.experimental.pallas import tpu as pltpu
from jax.experimental.pallas import tpu_sc as plsc

EPS = 1e-5
D = 128
_INV_D = 1.0 / D


_ROWS_BLK = 10000


def _ln_body(tab_ref, gamma_ref, beta_ref, out_ref):
    t = tab_ref[...]
    ones_over_d = jnp.full((D, D), _INV_D, dtype=jnp.float32)
    mean = jax.lax.dot(t, ones_over_d)
    m2 = jax.lax.dot(t * t, ones_over_d)
    var = m2 - mean * mean
    out_ref[...] = (t - mean) * lax.rsqrt(var + EPS) * gamma_ref[...] + beta_ref[...]


def _ln_table(table, gamma, beta):
    n_rows = table.shape[0]
    grid = n_rows // _ROWS_BLK
    return pl.pallas_call(
        _ln_body,
        grid=(grid,),
        in_specs=[
            pl.BlockSpec((_ROWS_BLK, D), lambda i: (i, 0)),
            pl.BlockSpec((1, D), lambda i: (0, 0)),
            pl.BlockSpec((1, D), lambda i: (0, 0)),
        ],
        out_specs=pl.BlockSpec((_ROWS_BLK, D), lambda i: (i, 0)),
        out_shape=jax.ShapeDtypeStruct((n_rows, D), jnp.float32),
    )(table, gamma.reshape(1, D), beta.reshape(1, D))


_NC, _NS = 2, 16
_NW = _NC * _NS
_K = 320
_NBUF = 3


def _make_gather(B):
    b_per_w = B // _NW
    nchunk = b_per_w // _K
    mesh = plsc.VectorSubcoreMesh(core_axis_name="c", subcore_axis_name="s")

    @functools.partial(
        pl.kernel,
        mesh=mesh,
        out_type=jax.ShapeDtypeStruct((B, D), jnp.float32),
        scratch_types=[
            pltpu.VMEM((b_per_w,), jnp.int32),
        ]
        + [pltpu.VMEM((_K, D), jnp.float32)] * _NBUF
        + [pltpu.SemaphoreType.DMA] * (2 * _NBUF),
    )
    def gather(tab_hbm, idx_hbm, out_hbm, idx_v, *scr):
        bufs = scr[:_NBUF]
        gsems = scr[_NBUF : 2 * _NBUF]
        osems = scr[2 * _NBUF :]
        wid = lax.axis_index("s") * _NC + lax.axis_index("c")
        base = wid * b_per_w
        pltpu.sync_copy(idx_hbm.at[pl.ds(base, b_per_w)], idx_v)
        h_in = [None] * _NBUF
        h_out = [None] * _NBUF
        for g in range(min(_NBUF - 1, nchunk)):
            h_in[g] = pltpu.async_copy(
                tab_hbm.at[idx_v.at[pl.ds(g * _K, _K)]], bufs[g], gsems[g]
            )
        for g in range(nchunk):
            b = g % _NBUF
            gn = g + _NBUF - 1
            if gn < nchunk:
                b2 = gn % _NBUF
                if h_out[b2] is not None:
                    h_out[b2].wait()
                h_in[b2] = pltpu.async_copy(
                    tab_hbm.at[idx_v.at[pl.ds(gn * _K, _K)]],
                    bufs[b2],
                    gsems[b2],
                )
            h_in[b].wait()
            h_out[b] = pltpu.async_copy(
                bufs[b], out_hbm.at[pl.ds(base + g * _K, _K)], osems[b]
            )
        for h in h_out:
            if h is not None:
                h.wait()

    return gather


def kernel(x, table, gamma, beta):
    batch, hist = x.shape
    B = batch * hist
    table_n = _ln_table(table, gamma, beta)
    idx_lmajor = x.T.reshape(B)
    out_flat = _make_gather(B)(table_n, idx_lmajor)
    return out_flat.reshape(hist, batch, D).transpose(1, 0, 2)
